# baseline (speedup 1.0000x reference)
import functools
import math

import jax
import jax.numpy as jnp
from jax import lax
from jax.experimental import pallas as pl
from jax.experimental.pallas import tpu as pltpu
from jax.experimental.pallas import tpu_sc as plsc

T = 4096
D = 768
E = 64
FF = 3072
FH = FF // 2
CAP = 64
TRASH = E * CAP
NTR = 34

NC = 2
NS = 16
NW = NC * NS
TPW = T // NW


def _gelu(x):
    c = math.sqrt(2.0 / math.pi)
    return x * 0.5 * (1.0 + jnp.tanh(c * (x + 0.044715 * x * x * x)))


def _router_body(x_ref, wr_ref, disp_ref, gate_ref, oh_ref, p_ref):
    x = x_ref[...]
    logits = jnp.dot(x, wr_ref[...], preferred_element_type=jnp.float32)
    m = jnp.max(logits, axis=1, keepdims=True)
    gate = 1.0 / jnp.sum(jnp.exp(logits - m), axis=1, keepdims=True)
    lane = lax.broadcasted_iota(jnp.int32, (T, E), 1).astype(jnp.float32)
    cand = jnp.where(logits == m, lane, 1e9)
    e_f = jnp.min(cand, axis=1, keepdims=True)
    onehot = (lane == e_f).astype(jnp.float32)
    oh_ref[...] = onehot

    CH = 128
    r = lax.broadcasted_iota(jnp.int32, (CH, CH), 0)
    c = lax.broadcasted_iota(jnp.int32, (CH, CH), 1)
    tri = (r >= c).astype(jnp.float32)

    def body(i, carry):
        mc = oh_ref[pl.ds(i * CH, CH), :]
        incl = jnp.dot(tri, mc, preferred_element_type=jnp.float32) + carry
        p_ref[pl.ds(i * CH, CH), :] = jnp.sum(incl * mc, axis=1, keepdims=True)
        return carry + jnp.sum(mc, axis=0, keepdims=True)

    lax.fori_loop(0, T // CH, body, jnp.zeros((1, E), jnp.float32))

    p = p_ref[...]
    keep = p < float(CAP)
    slot = e_f.astype(jnp.int32) * CAP + p.astype(jnp.int32) - 1
    disp_ref[...] = jnp.where(keep, slot, TRASH)
    gate_ref[...] = jnp.where(keep, gate, 0.0)


def _router(x, Wr):
    return pl.pallas_call(
        _router_body,
        out_shape=[
            jax.ShapeDtypeStruct((T, 1), jnp.int32),
            jax.ShapeDtypeStruct((T, 1), jnp.float32),
        ],
        scratch_shapes=[
            pltpu.VMEM((T, E), jnp.float32),
            pltpu.VMEM((T, 1), jnp.float32),
        ],
    )(x, Wr)


@functools.lru_cache(maxsize=None)
def _make_tables():
    mesh = plsc.VectorSubcoreMesh(core_axis_name="c", subcore_axis_name="s")

    @functools.partial(
        pl.kernel,
        out_type=(
            jax.ShapeDtypeStruct((NTR, 128), jnp.int32),
            jax.ShapeDtypeStruct((NTR, 128), jnp.float32),
        ),
        mesh=mesh,
        scratch_types=[
            pltpu.VMEM((T,), jnp.int32),
            pltpu.VMEM((T,), jnp.float32),
            pltpu.VMEM((NTR, 128), jnp.int32),
            pltpu.VMEM((NTR, 128), jnp.float32),
        ],
        compiler_params=pltpu.CompilerParams(needs_layout_passes=False),
    )
    def _tables(idx_hbm, gate_hbm, tok_hbm, gates_hbm, d_v, g_v, tok_v, gv_v):
        wid = lax.axis_index("s") * NC + lax.axis_index("c")

        @pl.when(wid == 0)
        def _build():
            pltpu.sync_copy(idx_hbm, d_v)
            pltpu.sync_copy(gate_hbm, g_v)

            def initb(i, carry):
                for j in range(128 // 16):
                    tok_v[i, pl.ds(j * 16, 16)] = jnp.full((16,), T, jnp.int32)
                return carry

            lax.fori_loop(0, NTR, initb, 0)

            def scat(rI, carry):
                idx16 = d_v[pl.ds(rI * 16, 16)]
                r16 = lax.shift_right_logical(idx16, 7)
                c16 = lax.bitwise_and(idx16, 127)
                t16 = lax.iota(jnp.int32, 16) + rI * 16
                plsc.store_scatter(tok_v, [r16, c16], t16)
                plsc.store_scatter(gv_v, [r16, c16], g_v[pl.ds(rI * 16, 16)])
                return carry

            lax.fori_loop(0, T // 16, scat, 0)
            pltpu.sync_copy(tok_v, tok_hbm)
            pltpu.sync_copy(gv_v, gates_hbm)

    return _tables


def _mlp_body(tok_ref, x_ref, w1_ref, b1_ref, w2_ref, b2_ref, gates_ref,
              out_ref, ei_s, acc_s):
    e = pl.program_id(0)
    f = pl.program_id(1)

    @pl.when(jnp.logical_and(e == 0, f == 0))
    def _zero():
        out_ref[...] = jnp.zeros_like(out_ref)

    @pl.when(f == 0)
    def _gather():
        def row(rI, carry):
            t = tok_ref[e * CAP + rI]
            tg = jnp.minimum(t, T - 1)
            ei_s[pl.ds(rI, 1), :] = x_ref[pl.ds(tg, 1), :]
            return carry

        lax.fori_loop(0, CAP, row, 0)

    ei = ei_s[...].astype(jnp.bfloat16)
    h = jnp.dot(ei, w1_ref[0].astype(jnp.bfloat16),
                preferred_element_type=jnp.float32)
    h = _gelu(h + b1_ref[0]).astype(jnp.bfloat16)
    o = jnp.dot(h, w2_ref[0].astype(jnp.bfloat16),
                preferred_element_type=jnp.float32)

    @pl.when(f == 0)
    def _acc0():
        acc_s[...] = o

    @pl.when(f == 1)
    def _combine():
        acc_s[...] = _gelu(acc_s[...] + o + b2_ref[0]) * gates_ref[0]

        def row(rI, carry):
            t = tok_ref[e * CAP + rI]

            @pl.when(t < T)
            def _store():
                out_ref[pl.ds(t, 1), :] = acc_s[pl.ds(rI, 1), :]

            return carry

        lax.fori_loop(0, CAP, row, 0)


def _mlp(tok, x, W1, b1, W2, b2, gates):
    grid_spec = pltpu.PrefetchScalarGridSpec(
        num_scalar_prefetch=1,
        grid=(E, 2),
        in_specs=[
            pl.BlockSpec((T, D), lambda e, f, tok: (0, 0)),
            pl.BlockSpec((1, D, FH), lambda e, f, tok: (e, 0, f)),
            pl.BlockSpec((1, 1, FH), lambda e, f, tok: (e, 0, f)),
            pl.BlockSpec((1, FH, D), lambda e, f, tok: (e, f, 0)),
            pl.BlockSpec((1, 1, D), lambda e, f, tok: (e, 0, 0)),
            pl.BlockSpec((1, CAP, 1), lambda e, f, tok: (e, 0, 0)),
        ],
        out_specs=pl.BlockSpec((T, D), lambda e, f, tok: (0, 0)),
        scratch_shapes=[
            pltpu.VMEM((CAP, D), jnp.float32),
            pltpu.VMEM((CAP, D), jnp.float32),
        ],
    )
    return pl.pallas_call(
        _mlp_body,
        grid_spec=grid_spec,
        out_shape=jax.ShapeDtypeStruct((T, D), jnp.float32),
    )(tok, x, W1, b1.reshape(E, 1, FF), W2, b2.reshape(E, 1, D),
      gates[: E * CAP].reshape(E, CAP, 1))


def kernel(inputs, Wr, W1, b1, W2, b2):
    x = inputs.reshape(T, D)
    disp_idx, gate = _router(x, Wr)
    tok, gates = _make_tables()(disp_idx.reshape(T), gate.reshape(T))
    out = _mlp(tok.reshape(NTR * 128), x, W1, b1, W2, b2,
               gates.reshape(NTR * 128))
    return out.reshape(inputs.shape)


# speedup vs baseline: 1.0629x; 1.0629x over previous
import functools
import math

import jax
import jax.numpy as jnp
from jax import lax
from jax.experimental import pallas as pl
from jax.experimental.pallas import tpu as pltpu
from jax.experimental.pallas import tpu_sc as plsc

T = 4096
D = 768
E = 64
FF = 3072
CAP = 64
NROWS = E * CAP + CAP
TRASH = E * CAP
NTR = 34

NC = 2
NS = 16
NW = NC * NS
TPW = T // NW


def _gelu(x):
    c = math.sqrt(2.0 / math.pi)
    return x * 0.5 * (1.0 + jnp.tanh(c * (x + 0.044715 * x * x * x)))


def _router_body(x_ref, wr_ref, disp_ref, gate_ref, oh_ref, p_ref):
    x = x_ref[...]
    logits = jnp.dot(x, wr_ref[...], preferred_element_type=jnp.float32)
    m = jnp.max(logits, axis=1, keepdims=True)
    gate = 1.0 / jnp.sum(jnp.exp(logits - m), axis=1, keepdims=True)
    lane = lax.broadcasted_iota(jnp.int32, (T, E), 1).astype(jnp.float32)
    cand = jnp.where(logits == m, lane, 1e9)
    e_f = jnp.min(cand, axis=1, keepdims=True)
    onehot = (lane == e_f).astype(jnp.float32)
    oh_ref[...] = onehot

    CH = 128
    r = lax.broadcasted_iota(jnp.int32, (CH, CH), 0)
    c = lax.broadcasted_iota(jnp.int32, (CH, CH), 1)
    tri = (r >= c).astype(jnp.float32)

    def body(i, carry):
        mc = oh_ref[pl.ds(i * CH, CH), :]
        incl = jnp.dot(tri, mc, preferred_element_type=jnp.float32) + carry
        p_ref[pl.ds(i * CH, CH), :] = jnp.sum(incl * mc, axis=1, keepdims=True)
        return carry + jnp.sum(mc, axis=0, keepdims=True)

    lax.fori_loop(0, T // CH, body, jnp.zeros((1, E), jnp.float32))

    p = p_ref[...]
    keep = p < float(CAP)
    slot = e_f.astype(jnp.int32) * CAP + p.astype(jnp.int32) - 1
    disp_ref[...] = jnp.where(keep, slot, TRASH)
    gate_ref[...] = jnp.where(keep, gate, 0.0)


def _router(x, Wr):
    return pl.pallas_call(
        _router_body,
        out_shape=[
            jax.ShapeDtypeStruct((T, 1), jnp.int32),
            jax.ShapeDtypeStruct((T, 1), jnp.float32),
        ],
        scratch_shapes=[
            pltpu.VMEM((T, E), jnp.float32),
            pltpu.VMEM((T, 1), jnp.float32),
        ],
    )(x, Wr)


@functools.lru_cache(maxsize=None)
def _make_dispatch():
    mesh = plsc.VectorSubcoreMesh(core_axis_name="c", subcore_axis_name="s")

    @functools.partial(
        pl.kernel,
        out_type=(
            jax.ShapeDtypeStruct((NROWS, D), jnp.float32),
            jax.ShapeDtypeStruct((NTR, 128), jnp.int32),
            jax.ShapeDtypeStruct((NTR, 128), jnp.float32),
        ),
        mesh=mesh,
        scratch_types=[
            pltpu.VMEM((TPW,), jnp.int32),
            pltpu.VMEM((TPW, D), jnp.float32),
            pltpu.VMEM((T,), jnp.int32),
            pltpu.VMEM((T,), jnp.float32),
            pltpu.VMEM((NTR, 128), jnp.int32),
            pltpu.VMEM((NTR, 128), jnp.float32),
            pltpu.SemaphoreType.DMA,
        ],
        compiler_params=pltpu.CompilerParams(needs_layout_passes=False),
    )
    def _dispatch(x_hbm, idx_hbm, gate_hbm, ei_hbm, tok_hbm, gates_hbm,
                  idx_v, rows_v, d_v, g_v, tok_v, gv_v, sem):
        wid = lax.axis_index("s") * NC + lax.axis_index("c")
        base = wid * TPW
        pltpu.sync_copy(idx_hbm.at[pl.ds(base, TPW)], idx_v)
        pltpu.sync_copy(x_hbm.at[pl.ds(base, TPW)], rows_v)
        cp = pltpu.async_copy(rows_v, ei_hbm.at[idx_v], sem)

        @pl.when(wid == 0)
        def _build():
            pltpu.sync_copy(idx_hbm, d_v)
            pltpu.sync_copy(gate_hbm, g_v)

            def initb(i, carry):
                for j in range(128 // 16):
                    tok_v[i, pl.ds(j * 16, 16)] = jnp.full((16,), T, jnp.int32)
                return carry

            lax.fori_loop(0, NTR, initb, 0)

            def scat(rI, carry):
                idx16 = d_v[pl.ds(rI * 16, 16)]
                r16 = lax.shift_right_logical(idx16, 7)
                c16 = lax.bitwise_and(idx16, 127)
                t16 = lax.iota(jnp.int32, 16) + rI * 16
                plsc.store_scatter(tok_v, [r16, c16], t16)
                plsc.store_scatter(gv_v, [r16, c16], g_v[pl.ds(rI * 16, 16)])
                return carry

            lax.fori_loop(0, T // 16, scat, 0)
            pltpu.sync_copy(tok_v, tok_hbm)
            pltpu.sync_copy(gv_v, gates_hbm)

        cp.wait()

    return _dispatch


def _mlp_body(tok_ref, ei_ref, w1_ref, b1_ref, w2_ref, b2_ref, gates_ref,
              out_ref, eo_s):
    e = pl.program_id(0)

    @pl.when(e == 0)
    def _zero():
        out_ref[...] = jnp.zeros_like(out_ref)

    ei = ei_ref[...].astype(jnp.bfloat16)
    h = jnp.dot(ei, w1_ref[0].astype(jnp.bfloat16),
                preferred_element_type=jnp.float32)
    h = _gelu(h + b1_ref[e]).astype(jnp.bfloat16)
    o = jnp.dot(h, w2_ref[0].astype(jnp.bfloat16),
                preferred_element_type=jnp.float32)
    eo_s[...] = _gelu(o + b2_ref[e]) * gates_ref[e]

    def row(rI, carry):
        t = tok_ref[e * CAP + rI]

        @pl.when(t < T)
        def _store():
            out_ref[pl.ds(t, 1), :] = eo_s[pl.ds(rI, 1), :]

        return carry

    lax.fori_loop(0, CAP, row, 0)


def _mlp(tok, ei, W1, b1, W2, b2, gates):
    grid_spec = pltpu.PrefetchScalarGridSpec(
        num_scalar_prefetch=1,
        grid=(E,),
        in_specs=[
            pl.BlockSpec((CAP, D), lambda e, tok: (e, 0)),
            pl.BlockSpec((1, D, FF), lambda e, tok: (e, 0, 0)),
            pl.BlockSpec((E, 1, FF), lambda e, tok: (0, 0, 0)),
            pl.BlockSpec((1, FF, D), lambda e, tok: (e, 0, 0)),
            pl.BlockSpec((E, 1, D), lambda e, tok: (0, 0, 0)),
            pl.BlockSpec((E, CAP, 1), lambda e, tok: (0, 0, 0)),
        ],
        out_specs=pl.BlockSpec((T, D), lambda e, tok: (0, 0)),
        scratch_shapes=[pltpu.VMEM((CAP, D), jnp.float32)],
    )
    return pl.pallas_call(
        _mlp_body,
        grid_spec=grid_spec,
        out_shape=jax.ShapeDtypeStruct((T, D), jnp.float32),
    )(tok, ei, W1, b1.reshape(E, 1, FF), W2, b2.reshape(E, 1, D),
      gates[: E * CAP].reshape(E, CAP, 1))


def kernel(inputs, Wr, W1, b1, W2, b2):
    x = inputs.reshape(T, D)
    disp_idx, gate = _router(x, Wr)
    ei, tok, gates = _make_dispatch()(x, disp_idx.reshape(T), gate.reshape(T))
    out = _mlp(tok.reshape(NTR * 128), ei, W1, b1, W2, b2,
               gates.reshape(NTR * 128))
    return out.reshape(inputs.shape)


# speedup vs baseline: 1.0794x; 1.0155x over previous
import functools
import math

import jax
import jax.numpy as jnp
from jax import lax
from jax.experimental import pallas as pl
from jax.experimental.pallas import tpu as pltpu
from jax.experimental.pallas import tpu_sc as plsc

T = 4096
D = 768
E = 64
FF = 3072
CAP = 64
NROWS = E * CAP + CAP
TRASH = E * CAP
NTR = 34

NC = 2
NS = 16
NW = NC * NS
TPW = T // NW


def _gelu(x):
    c = math.sqrt(2.0 / math.pi)
    return x * 0.5 * (1.0 + jnp.tanh(c * (x + 0.044715 * x * x * x)))


def _router_body(x_ref, wr_ref, disp_ref, tok_ref, gates_ref, oh_ref, p_ref):
    x = x_ref[...]
    logits = jnp.dot(x, wr_ref[...], preferred_element_type=jnp.float32)
    m = jnp.max(logits, axis=1, keepdims=True)
    gate = 1.0 / jnp.sum(jnp.exp(logits - m), axis=1, keepdims=True)
    lane = lax.broadcasted_iota(jnp.int32, (T, E), 1).astype(jnp.float32)
    cand = jnp.where(logits == m, lane, 1e9)
    e_f = jnp.min(cand, axis=1, keepdims=True)
    onehot = (lane == e_f).astype(jnp.float32)
    oh_ref[...] = onehot

    CH = 128
    r = lax.broadcasted_iota(jnp.int32, (CH, CH), 0)
    c = lax.broadcasted_iota(jnp.int32, (CH, CH), 1)
    tri = (r >= c).astype(jnp.float32)

    def body(i, carry):
        mc = oh_ref[pl.ds(i * CH, CH), :]
        incl = jnp.dot(tri, mc, preferred_element_type=jnp.float32) + carry
        p_ref[pl.ds(i * CH, CH), :] = jnp.sum(incl * mc, axis=1, keepdims=True)
        return carry + jnp.sum(mc, axis=0, keepdims=True)

    lax.fori_loop(0, T // CH, body, jnp.zeros((1, E), jnp.float32))

    p = p_ref[...]
    keep = p < float(CAP)
    slot = e_f.astype(jnp.int32) * CAP + p.astype(jnp.int32) - 1
    disp_ref[...] = jnp.where(keep, slot, TRASH)

    cslot = jnp.where(keep, p.astype(jnp.int32) - 1, 2 * CAP)
    pos1h = (lax.broadcasted_iota(jnp.int32, (T, CAP), 1) == cslot)
    pos1h = pos1h.astype(jnp.float32)
    tvals = lax.broadcasted_iota(jnp.int32, (T, 1), 0).astype(jnp.float32)
    dn = (((0,), (0,)), ((), ()))
    tok_f = lax.dot_general(onehot * tvals, pos1h, dn,
                            preferred_element_type=jnp.float32)
    cnt = lax.dot_general(onehot, pos1h, dn,
                          preferred_element_type=jnp.float32)
    gat = lax.dot_general(onehot * gate, pos1h, dn,
                          preferred_element_type=jnp.float32)
    tok_ref[...] = jnp.where(cnt > 0.0, tok_f.astype(jnp.int32), T)
    gates_ref[...] = gat


def _router(x, Wr):
    return pl.pallas_call(
        _router_body,
        out_shape=[
            jax.ShapeDtypeStruct((T, 1), jnp.int32),
            jax.ShapeDtypeStruct((E, CAP), jnp.int32),
            jax.ShapeDtypeStruct((E, CAP), jnp.float32),
        ],
        scratch_shapes=[
            pltpu.VMEM((T, E), jnp.float32),
            pltpu.VMEM((T, 1), jnp.float32),
        ],
    )(x, Wr)


@functools.lru_cache(maxsize=None)
def _make_dispatch():
    mesh = plsc.VectorSubcoreMesh(core_axis_name="c", subcore_axis_name="s")

    @functools.partial(
        pl.kernel,
        out_type=jax.ShapeDtypeStruct((NROWS, D), jnp.float32),
        mesh=mesh,
        scratch_types=[
            pltpu.VMEM((TPW,), jnp.int32),
            pltpu.VMEM((TPW, D), jnp.float32),
            pltpu.SemaphoreType.DMA,
        ],
        compiler_params=pltpu.CompilerParams(needs_layout_passes=False),
    )
    def _dispatch(x_hbm, idx_hbm, ei_hbm, idx_v, rows_v, sem):
        wid = lax.axis_index("s") * NC + lax.axis_index("c")
        base = wid * TPW
        pltpu.sync_copy(idx_hbm.at[pl.ds(base, TPW)], idx_v)
        pltpu.sync_copy(x_hbm.at[pl.ds(base, TPW)], rows_v)
        pltpu.async_copy(rows_v, ei_hbm.at[idx_v], sem).wait()

    return _dispatch


def _mlp_body(tok_ref, ei_ref, w1_ref, b1_ref, w2_ref, b2_ref, gates_ref,
              out_ref, eo_s):
    e = pl.program_id(0)

    @pl.when(e == 0)
    def _zero():
        out_ref[...] = jnp.zeros_like(out_ref)

    ei = ei_ref[...].astype(jnp.bfloat16)
    h = jnp.dot(ei, w1_ref[0].astype(jnp.bfloat16),
                preferred_element_type=jnp.float32)
    h = _gelu(h + b1_ref[e]).astype(jnp.bfloat16)
    o = jnp.dot(h, w2_ref[0].astype(jnp.bfloat16),
                preferred_element_type=jnp.float32)
    eo_s[...] = _gelu(o + b2_ref[e]) * gates_ref[e]

    def row(rI, carry):
        t = tok_ref[e * CAP + rI]

        @pl.when(t < T)
        def _store():
            out_ref[pl.ds(t, 1), :] = eo_s[pl.ds(rI, 1), :]

        return carry

    lax.fori_loop(0, CAP, row, 0)


def _mlp(tok, ei, W1, b1, W2, b2, gates):
    grid_spec = pltpu.PrefetchScalarGridSpec(
        num_scalar_prefetch=1,
        grid=(E,),
        in_specs=[
            pl.BlockSpec((CAP, D), lambda e, tok: (e, 0)),
            pl.BlockSpec((1, D, FF), lambda e, tok: (e, 0, 0)),
            pl.BlockSpec((E, 1, FF), lambda e, tok: (0, 0, 0)),
            pl.BlockSpec((1, FF, D), lambda e, tok: (e, 0, 0)),
            pl.BlockSpec((E, 1, D), lambda e, tok: (0, 0, 0)),
            pl.BlockSpec((E, CAP, 1), lambda e, tok: (0, 0, 0)),
        ],
        out_specs=pl.BlockSpec((T, D), lambda e, tok: (0, 0)),
        scratch_shapes=[pltpu.VMEM((CAP, D), jnp.float32)],
    )
    return pl.pallas_call(
        _mlp_body,
        grid_spec=grid_spec,
        out_shape=jax.ShapeDtypeStruct((T, D), jnp.float32),
    )(tok, ei, W1, b1.reshape(E, 1, FF), W2, b2.reshape(E, 1, D),
      gates.reshape(E, CAP, 1))


def kernel(inputs, Wr, W1, b1, W2, b2):
    x = inputs.reshape(T, D)
    disp_idx, tok, gates = _router(x, Wr)
    ei = _make_dispatch()(x, disp_idx.reshape(T))
    out = _mlp(tok.reshape(E * CAP), ei, W1, b1, W2, b2, gates)
    return out.reshape(inputs.shape)
